# Initial kernel scaffold; baseline (speedup 1.0000x reference)
import functools

import jax
import jax.numpy as jnp
from jax import lax
from jax.experimental import pallas as pl
from jax.experimental.pallas import tpu as pltpu
from jax.experimental.pallas import tpu_sc as plsc

N = 10000
E = 320000
D = 128

NUM_TILES = 16
ROWS_PER_TILE = N // NUM_TILES
EDGES_PER_TILE = E // NUM_TILES
CHUNK = 80
NUM_CHUNKS = EDGES_PER_TILE // CHUNK

_MESH = plsc.VectorSubcoreMesh(core_axis_name="c", subcore_axis_name="s")


@functools.partial(
    pl.kernel,
    mesh=_MESH,
    out_type=[
        jax.ShapeDtypeStruct((N, D), jnp.float32),
        jax.ShapeDtypeStruct((N, D), jnp.float32),
    ],
    scratch_types=[
        pltpu.VMEM((CHUNK,), jnp.int32),
        pltpu.VMEM((CHUNK,), jnp.int32),
        pltpu.VMEM((CHUNK, D), jnp.float32),
        pltpu.VMEM_SHARED((N, D), jnp.float32),
        pltpu.SemaphoreType.DMA,
    ],
)
def _segsum_sc(x1_hbm, src1_hbm, dst1_hbm, x2_hbm, src2_hbm, dst2_hbm,
               zeros_hbm, acc1_hbm, acc2_hbm,
               src_v, dst_v, rows_v, acc_sh, sem):
    c = lax.axis_index("c")
    s = lax.axis_index("s")
    row0 = s * ROWS_PER_TILE

    pltpu.sync_copy(zeros_hbm, acc_sh.at[pl.ds(row0, ROWS_PER_TILE)])
    plsc.subcore_barrier()

    def _edge_loop(x_hbm, src_hbm, dst_hbm):
        base = s * EDGES_PER_TILE

        def body(i, carry):
            off = base + i * CHUNK
            pltpu.sync_copy(src_hbm.at[pl.ds(off, CHUNK)], src_v)
            pltpu.sync_copy(dst_hbm.at[pl.ds(off, CHUNK)], dst_v)
            pltpu.async_copy(x_hbm.at[src_v], rows_v, sem).wait()
            pltpu.sync_copy(rows_v, acc_sh.at[dst_v], add=True)
            return carry

        lax.fori_loop(0, NUM_CHUNKS, body, 0)

    @pl.when(c == 0)
    def _():
        _edge_loop(x1_hbm, src1_hbm, dst1_hbm)

    @pl.when(c == 1)
    def _():
        _edge_loop(x2_hbm, src2_hbm, dst2_hbm)

    plsc.subcore_barrier()

    @pl.when(c == 0)
    def _():
        pltpu.sync_copy(acc_sh.at[pl.ds(row0, ROWS_PER_TILE)],
                        acc1_hbm.at[pl.ds(row0, ROWS_PER_TILE)])

    @pl.when(c == 1)
    def _():
        pltpu.sync_copy(acc_sh.at[pl.ds(row0, ROWS_PER_TILE)],
                        acc2_hbm.at[pl.ds(row0, ROWS_PER_TILE)])


@functools.partial(
    pl.kernel,
    mesh=_MESH,
    out_type=[
        jax.ShapeDtypeStruct((N, 16), jnp.float32),
        jax.ShapeDtypeStruct((N, 16), jnp.float32),
    ],
    scratch_types=[
        pltpu.VMEM((CHUNK,), jnp.int32),
        pltpu.VMEM((CHUNK, 16), jnp.float32),
        pltpu.VMEM_SHARED((N, 16), jnp.float32),
    ],
)
def _degree_sc(dst1_hbm, dst2_hbm, ones_hbm, zeros_hbm,
               deg1_hbm, deg2_hbm, dst_v, ones_v, deg_sh):
    c = lax.axis_index("c")
    s = lax.axis_index("s")
    row0 = s * ROWS_PER_TILE

    pltpu.sync_copy(ones_hbm, ones_v)
    pltpu.sync_copy(zeros_hbm, deg_sh.at[pl.ds(row0, ROWS_PER_TILE)])
    plsc.subcore_barrier()

    def _edge_loop(dst_hbm):
        base = s * EDGES_PER_TILE

        def body(i, carry):
            off = base + i * CHUNK
            pltpu.sync_copy(dst_hbm.at[pl.ds(off, CHUNK)], dst_v)
            pltpu.sync_copy(ones_v, deg_sh.at[dst_v], add=True)
            return carry

        lax.fori_loop(0, NUM_CHUNKS, body, 0)

    @pl.when(c == 0)
    def _():
        _edge_loop(dst1_hbm)

    @pl.when(c == 1)
    def _():
        _edge_loop(dst2_hbm)

    plsc.subcore_barrier()

    @pl.when(c == 0)
    def _():
        pltpu.sync_copy(deg_sh.at[pl.ds(row0, ROWS_PER_TILE)],
                        deg1_hbm.at[pl.ds(row0, ROWS_PER_TILE)])

    @pl.when(c == 1)
    def _():
        pltpu.sync_copy(deg_sh.at[pl.ds(row0, ROWS_PER_TILE)],
                        deg2_hbm.at[pl.ds(row0, ROWS_PER_TILE)])


_TC_BLOCK = 2000


def _tc_layer_body(x_ref, acc_ref, deg_ref, w_ref, b_ref, o_ref, *, last):
    t = jnp.dot(acc_ref[...], w_ref[...],
                preferred_element_type=jnp.float32,
                precision=lax.Precision.HIGHEST)
    t = t + deg_ref[:, 0:1] * b_ref[...]
    t = x_ref[...] + jnp.maximum(t, 0.0)
    if last:
        nrm = jnp.sqrt(jnp.sum(t * t, axis=1, keepdims=True))
        t = t / jnp.maximum(nrm, 1e-12)
    o_ref[...] = t


def _tc_layer(x, acc, deg, W, b2d, last):
    grid = (N // _TC_BLOCK,)
    return pl.pallas_call(
        functools.partial(_tc_layer_body, last=last),
        grid=grid,
        in_specs=[
            pl.BlockSpec((_TC_BLOCK, D), lambda i: (i, 0)),
            pl.BlockSpec((_TC_BLOCK, D), lambda i: (i, 0)),
            pl.BlockSpec((_TC_BLOCK, 16), lambda i: (i, 0)),
            pl.BlockSpec((D, D), lambda i: (0, 0)),
            pl.BlockSpec((1, D), lambda i: (0, 0)),
        ],
        out_specs=pl.BlockSpec((_TC_BLOCK, D), lambda i: (i, 0)),
        out_shape=jax.ShapeDtypeStruct((N, D), jnp.float32),
    )(x, acc, deg, W, b2d)


def kernel(x1, edge_index1, x2, edge_index2,
           W0, b0, W1, b1, W2, b2, W3, b3):
    src1 = edge_index1[0].astype(jnp.int32)
    dst1 = edge_index1[1].astype(jnp.int32)
    src2 = edge_index2[0].astype(jnp.int32)
    dst2 = edge_index2[1].astype(jnp.int32)

    zeros128 = jnp.zeros((ROWS_PER_TILE, D), jnp.float32)
    zeros16 = jnp.zeros((ROWS_PER_TILE, 16), jnp.float32)
    ones16 = jnp.ones((CHUNK, 16), jnp.float32)

    deg1, deg2 = _degree_sc(dst1, dst2, ones16, zeros16)

    params = [(W0, b0), (W1, b1), (W2, b2), (W3, b3)]
    for layer, (W, b) in enumerate(params):
        acc1, acc2 = _segsum_sc(x1, src1, dst1, x2, src2, dst2, zeros128)
        last = layer == len(params) - 1
        b2d = b.reshape(1, D)
        x1 = _tc_layer(x1, acc1, deg1, W, b2d, last)
        x2 = _tc_layer(x2, acc2, deg2, W, b2d, last)
    return (x1, x2)


# speedup vs baseline: 3.1111x; 3.1111x over previous
import functools

import jax
import jax.numpy as jnp
from jax import lax
from jax.experimental import pallas as pl
from jax.experimental.pallas import tpu as pltpu
from jax.experimental.pallas import tpu_sc as plsc

N = 10000
E = 320000
D = 128

NUM_TILES = 16
NPAD = 10240
ROWS_PER_TILE = NPAD // NUM_TILES
EDGES_PER_TILE = E // NUM_TILES
CHUNK = 80
NUM_CHUNKS = EDGES_PER_TILE // CHUNK

_MESH = plsc.VectorSubcoreMesh(core_axis_name="c", subcore_axis_name="s")


@functools.partial(
    pl.kernel,
    mesh=_MESH,
    out_type=[
        jax.ShapeDtypeStruct((NPAD, D), jnp.float32),
        jax.ShapeDtypeStruct((NPAD, D), jnp.float32),
    ],
    scratch_types=[
        pltpu.VMEM((CHUNK,), jnp.int32),
        pltpu.VMEM((CHUNK,), jnp.int32),
        pltpu.VMEM((CHUNK, D), jnp.float32),
        pltpu.VMEM_SHARED((NPAD, D), jnp.float32),
        pltpu.SemaphoreType.DMA,
    ],
)
def _segsum_sc(x1_hbm, src1_hbm, dst1_hbm, x2_hbm, src2_hbm, dst2_hbm,
               zeros_hbm, acc1_hbm, acc2_hbm,
               src_v, dst_v, rows_v, acc_sh, sem):
    c = lax.axis_index("c")
    s = lax.axis_index("s")
    row0 = s * ROWS_PER_TILE

    pltpu.sync_copy(zeros_hbm, acc_sh.at[pl.ds(row0, ROWS_PER_TILE)])
    plsc.subcore_barrier()

    def _edge_loop(x_hbm, src_hbm, dst_hbm):
        base = s * EDGES_PER_TILE

        def body(i, carry):
            off = base + i * CHUNK
            pltpu.sync_copy(src_hbm.at[pl.ds(off, CHUNK)], src_v)
            pltpu.sync_copy(dst_hbm.at[pl.ds(off, CHUNK)], dst_v)
            pltpu.async_copy(x_hbm.at[src_v], rows_v, sem).wait()
            pltpu.sync_copy(rows_v, acc_sh.at[dst_v], add=True)
            return carry

        lax.fori_loop(0, NUM_CHUNKS, body, 0)

    @pl.when(c == 0)
    def _():
        _edge_loop(x1_hbm, src1_hbm, dst1_hbm)

    @pl.when(c == 1)
    def _():
        _edge_loop(x2_hbm, src2_hbm, dst2_hbm)

    plsc.subcore_barrier()

    @pl.when(c == 0)
    def _():
        pltpu.sync_copy(acc_sh.at[pl.ds(row0, ROWS_PER_TILE)],
                        acc1_hbm.at[pl.ds(row0, ROWS_PER_TILE)])

    @pl.when(c == 1)
    def _():
        pltpu.sync_copy(acc_sh.at[pl.ds(row0, ROWS_PER_TILE)],
                        acc2_hbm.at[pl.ds(row0, ROWS_PER_TILE)])


@functools.partial(
    pl.kernel,
    mesh=_MESH,
    out_type=[
        jax.ShapeDtypeStruct((NPAD, 16), jnp.float32),
        jax.ShapeDtypeStruct((NPAD, 16), jnp.float32),
    ],
    scratch_types=[
        pltpu.VMEM((CHUNK,), jnp.int32),
        pltpu.VMEM((CHUNK, 16), jnp.float32),
        pltpu.VMEM_SHARED((NPAD, 16), jnp.float32),
    ],
)
def _degree_sc(dst1_hbm, dst2_hbm, ones_hbm, zeros_hbm,
               deg1_hbm, deg2_hbm, dst_v, ones_v, deg_sh):
    c = lax.axis_index("c")
    s = lax.axis_index("s")
    row0 = s * ROWS_PER_TILE

    pltpu.sync_copy(ones_hbm, ones_v)
    pltpu.sync_copy(zeros_hbm, deg_sh.at[pl.ds(row0, ROWS_PER_TILE)])
    plsc.subcore_barrier()

    def _edge_loop(dst_hbm):
        base = s * EDGES_PER_TILE

        def body(i, carry):
            off = base + i * CHUNK
            pltpu.sync_copy(dst_hbm.at[pl.ds(off, CHUNK)], dst_v)
            pltpu.sync_copy(ones_v, deg_sh.at[dst_v], add=True)
            return carry

        lax.fori_loop(0, NUM_CHUNKS, body, 0)

    @pl.when(c == 0)
    def _():
        _edge_loop(dst1_hbm)

    @pl.when(c == 1)
    def _():
        _edge_loop(dst2_hbm)

    plsc.subcore_barrier()

    @pl.when(c == 0)
    def _():
        pltpu.sync_copy(deg_sh.at[pl.ds(row0, ROWS_PER_TILE)],
                        deg1_hbm.at[pl.ds(row0, ROWS_PER_TILE)])

    @pl.when(c == 1)
    def _():
        pltpu.sync_copy(deg_sh.at[pl.ds(row0, ROWS_PER_TILE)],
                        deg2_hbm.at[pl.ds(row0, ROWS_PER_TILE)])


_TC_BLOCK = 2000


def _tc_layer_body(x_ref, acc_ref, deg_ref, w_ref, b_ref, o_ref, *, last):
    t = jnp.dot(acc_ref[...], w_ref[...],
                preferred_element_type=jnp.float32,
                precision=lax.Precision.HIGHEST)
    t = t + deg_ref[:, 0:1] * b_ref[...]
    t = x_ref[...] + jnp.maximum(t, 0.0)
    if last:
        nrm = jnp.sqrt(jnp.sum(t * t, axis=1, keepdims=True))
        t = t / jnp.maximum(nrm, 1e-12)
    o_ref[...] = t


def _tc_layer(x, acc, deg, W, b2d, last):
    grid = (N // _TC_BLOCK,)
    return pl.pallas_call(
        functools.partial(_tc_layer_body, last=last),
        grid=grid,
        in_specs=[
            pl.BlockSpec((_TC_BLOCK, D), lambda i: (i, 0)),
            pl.BlockSpec((_TC_BLOCK, D), lambda i: (i, 0)),
            pl.BlockSpec((_TC_BLOCK, 16), lambda i: (i, 0)),
            pl.BlockSpec((D, D), lambda i: (0, 0)),
            pl.BlockSpec((1, D), lambda i: (0, 0)),
        ],
        out_specs=pl.BlockSpec((_TC_BLOCK, D), lambda i: (i, 0)),
        out_shape=jax.ShapeDtypeStruct((N, D), jnp.float32),
    )(x, acc, deg, W, b2d)


def kernel(x1, edge_index1, x2, edge_index2,
           W0, b0, W1, b1, W2, b2, W3, b3):
    src1 = edge_index1[0].astype(jnp.int32)
    dst1 = edge_index1[1].astype(jnp.int32)
    src2 = edge_index2[0].astype(jnp.int32)
    dst2 = edge_index2[1].astype(jnp.int32)

    zeros128 = jnp.zeros((ROWS_PER_TILE, D), jnp.float32)
    zeros16 = jnp.zeros((ROWS_PER_TILE, 16), jnp.float32)
    ones16 = jnp.ones((CHUNK, 16), jnp.float32)

    deg1, deg2 = _degree_sc(dst1, dst2, ones16, zeros16)

    params = [(W0, b0), (W1, b1), (W2, b2), (W3, b3)]
    for layer, (W, b) in enumerate(params):
        acc1, acc2 = _segsum_sc(x1, src1, dst1, x2, src2, dst2, zeros128)
        last = layer == len(params) - 1
        b2d = b.reshape(1, D)
        x1 = _tc_layer(x1, acc1, deg1, W, b2d, last)
        x2 = _tc_layer(x2, acc2, deg2, W, b2d, last)
    return (x1, x2)


# speedup vs baseline: 3.3195x; 1.0670x over previous
import functools

import jax
import jax.numpy as jnp
from jax import lax
from jax.experimental import pallas as pl
from jax.experimental.pallas import tpu as pltpu
from jax.experimental.pallas import tpu_sc as plsc

N = 10000
E = 320000
D = 128

NUM_TILES = 16
NPAD = 10240
ROWS_PER_TILE = NPAD // NUM_TILES
EDGES_PER_TILE = E // NUM_TILES
CHUNK = 128
NUM_CHUNKS = 160
PAD_EDGES = NUM_CHUNKS * CHUNK - EDGES_PER_TILE
NUM_PAIRS = NUM_CHUNKS // 2
DUMMY_DST = NPAD - 1

_MESH = plsc.VectorSubcoreMesh(core_axis_name="c", subcore_axis_name="s")


@functools.partial(
    pl.kernel,
    mesh=_MESH,
    out_type=[
        jax.ShapeDtypeStruct((NPAD, D), jnp.float32),
        jax.ShapeDtypeStruct((NPAD, D), jnp.float32),
    ],
    scratch_types=[
        pltpu.VMEM((2, 2, CHUNK), jnp.int32),
        pltpu.VMEM((2, 2, CHUNK), jnp.int32),
        pltpu.VMEM((CHUNK, D), jnp.float32),
        pltpu.VMEM((CHUNK, D), jnp.float32),
        pltpu.VMEM_SHARED((NPAD, D), jnp.float32),
        pltpu.SemaphoreType.DMA,
        pltpu.SemaphoreType.DMA,
        pltpu.SemaphoreType.DMA,
        pltpu.SemaphoreType.DMA,
    ],
)
def _segsum_sc(x1_hbm, src1_hbm, dst1_hbm, x2_hbm, src2_hbm, dst2_hbm,
               zeros_hbm, acc1_hbm, acc2_hbm,
               sidx, didx, buf0, buf1, acc_sh, sem0, sem1, semi0, semi1):
    c = lax.axis_index("c")
    s = lax.axis_index("s")
    row0 = s * ROWS_PER_TILE

    pltpu.sync_copy(zeros_hbm, acc_sh.at[pl.ds(row0, ROWS_PER_TILE)])
    plsc.subcore_barrier()

    def _edge_loop(x_hbm, src_hbm, dst_hbm):
        base = s * NUM_CHUNKS * CHUNK

        def start_idx(g, p, semi):
            off = base + g * (2 * CHUNK)
            pltpu.async_copy(src_hbm.at[pl.ds(off, CHUNK)], sidx.at[p, 0], semi)
            pltpu.async_copy(src_hbm.at[pl.ds(off + CHUNK, CHUNK)],
                             sidx.at[p, 1], semi)
            pltpu.async_copy(dst_hbm.at[pl.ds(off, CHUNK)], didx.at[p, 0], semi)
            pltpu.async_copy(dst_hbm.at[pl.ds(off + CHUNK, CHUNK)],
                             didx.at[p, 1], semi)

        def wait_idx(g, p, semi):
            off = base + g * (2 * CHUNK)
            pltpu.make_async_copy(src_hbm.at[pl.ds(off, CHUNK)],
                                  sidx.at[p, 0], semi).wait()
            pltpu.make_async_copy(src_hbm.at[pl.ds(off + CHUNK, CHUNK)],
                                  sidx.at[p, 1], semi).wait()
            pltpu.make_async_copy(dst_hbm.at[pl.ds(off, CHUNK)],
                                  didx.at[p, 0], semi).wait()
            pltpu.make_async_copy(dst_hbm.at[pl.ds(off + CHUNK, CHUNK)],
                                  didx.at[p, 1], semi).wait()

        def do_pair(g, p, semi, semi_next):
            wait_idx(g, p, semi)
            pltpu.async_copy(x_hbm.at[sidx.at[p, 0]], buf0, sem0)
            pltpu.async_copy(x_hbm.at[sidx.at[p, 1]], buf1, sem1)

            @pl.when(g + 1 < NUM_PAIRS)
            def _():
                start_idx(g + 1, 1 - p, semi_next)

            pltpu.make_async_copy(x_hbm.at[sidx.at[p, 0]], buf0, sem0).wait()
            pltpu.sync_copy(buf0, acc_sh.at[didx.at[p, 0]], add=True)
            pltpu.make_async_copy(x_hbm.at[sidx.at[p, 1]], buf1, sem1).wait()
            pltpu.sync_copy(buf1, acc_sh.at[didx.at[p, 1]], add=True)

        start_idx(0, 0, semi0)

        def body(q, carry):
            do_pair(2 * q, 0, semi0, semi1)
            do_pair(2 * q + 1, 1, semi1, semi0)
            return carry

        lax.fori_loop(0, NUM_PAIRS // 2, body, 0)

    @pl.when(c == 0)
    def _():
        _edge_loop(x1_hbm, src1_hbm, dst1_hbm)

    @pl.when(c == 1)
    def _():
        _edge_loop(x2_hbm, src2_hbm, dst2_hbm)

    plsc.subcore_barrier()

    @pl.when(c == 0)
    def _():
        pltpu.sync_copy(acc_sh.at[pl.ds(row0, ROWS_PER_TILE)],
                        acc1_hbm.at[pl.ds(row0, ROWS_PER_TILE)])

    @pl.when(c == 1)
    def _():
        pltpu.sync_copy(acc_sh.at[pl.ds(row0, ROWS_PER_TILE)],
                        acc2_hbm.at[pl.ds(row0, ROWS_PER_TILE)])


@functools.partial(
    pl.kernel,
    mesh=_MESH,
    out_type=[
        jax.ShapeDtypeStruct((NPAD, 16), jnp.float32),
        jax.ShapeDtypeStruct((NPAD, 16), jnp.float32),
    ],
    scratch_types=[
        pltpu.VMEM((CHUNK,), jnp.int32),
        pltpu.VMEM((CHUNK, 16), jnp.float32),
        pltpu.VMEM_SHARED((NPAD, 16), jnp.float32),
    ],
)
def _degree_sc(dst1_hbm, dst2_hbm, ones_hbm, zeros_hbm,
               deg1_hbm, deg2_hbm, dst_v, ones_v, deg_sh):
    c = lax.axis_index("c")
    s = lax.axis_index("s")
    row0 = s * ROWS_PER_TILE

    pltpu.sync_copy(ones_hbm, ones_v)
    pltpu.sync_copy(zeros_hbm, deg_sh.at[pl.ds(row0, ROWS_PER_TILE)])
    plsc.subcore_barrier()

    def _edge_loop(dst_hbm):
        base = s * NUM_CHUNKS * CHUNK

        def body(i, carry):
            pltpu.sync_copy(dst_hbm.at[pl.ds(base + i * CHUNK, CHUNK)], dst_v)
            pltpu.sync_copy(ones_v, deg_sh.at[dst_v], add=True)
            return carry

        lax.fori_loop(0, NUM_CHUNKS, body, 0)

    @pl.when(c == 0)
    def _():
        _edge_loop(dst1_hbm)

    @pl.when(c == 1)
    def _():
        _edge_loop(dst2_hbm)

    plsc.subcore_barrier()

    @pl.when(c == 0)
    def _():
        pltpu.sync_copy(deg_sh.at[pl.ds(row0, ROWS_PER_TILE)],
                        deg1_hbm.at[pl.ds(row0, ROWS_PER_TILE)])

    @pl.when(c == 1)
    def _():
        pltpu.sync_copy(deg_sh.at[pl.ds(row0, ROWS_PER_TILE)],
                        deg2_hbm.at[pl.ds(row0, ROWS_PER_TILE)])


_TC_BLOCK = 2000


def _tc_layer_body(x_ref, acc_ref, deg_ref, w_ref, b_ref, o_ref, *, last):
    t = jnp.dot(acc_ref[...], w_ref[...],
                preferred_element_type=jnp.float32,
                precision=lax.Precision.HIGHEST)
    t = t + deg_ref[:, 0:1] * b_ref[...]
    t = x_ref[...] + jnp.maximum(t, 0.0)
    if last:
        nrm = jnp.sqrt(jnp.sum(t * t, axis=1, keepdims=True))
        t = t / jnp.maximum(nrm, 1e-12)
    o_ref[...] = t


def _tc_layer(x, acc, deg, W, b2d, last):
    grid = (N // _TC_BLOCK,)
    return pl.pallas_call(
        functools.partial(_tc_layer_body, last=last),
        grid=grid,
        in_specs=[
            pl.BlockSpec((_TC_BLOCK, D), lambda i: (i, 0)),
            pl.BlockSpec((_TC_BLOCK, D), lambda i: (i, 0)),
            pl.BlockSpec((_TC_BLOCK, 16), lambda i: (i, 0)),
            pl.BlockSpec((D, D), lambda i: (0, 0)),
            pl.BlockSpec((1, D), lambda i: (0, 0)),
        ],
        out_specs=pl.BlockSpec((_TC_BLOCK, D), lambda i: (i, 0)),
        out_shape=jax.ShapeDtypeStruct((N, D), jnp.float32),
    )(x, acc, deg, W, b2d)


def kernel(x1, edge_index1, x2, edge_index2,
           W0, b0, W1, b1, W2, b2, W3, b3):
    def _pad_idx(v, fill):
        t = v.astype(jnp.int32).reshape(NUM_TILES, EDGES_PER_TILE)
        t = jnp.pad(t, ((0, 0), (0, PAD_EDGES)), constant_values=fill)
        return t.reshape(NUM_TILES * NUM_CHUNKS * CHUNK)

    src1 = _pad_idx(edge_index1[0], 0)
    dst1 = _pad_idx(edge_index1[1], DUMMY_DST)
    src2 = _pad_idx(edge_index2[0], 0)
    dst2 = _pad_idx(edge_index2[1], DUMMY_DST)

    zeros128 = jnp.zeros((ROWS_PER_TILE, D), jnp.float32)
    zeros16 = jnp.zeros((ROWS_PER_TILE, 16), jnp.float32)
    ones16 = jnp.ones((CHUNK, 16), jnp.float32)

    deg1, deg2 = _degree_sc(dst1, dst2, ones16, zeros16)

    params = [(W0, b0), (W1, b1), (W2, b2), (W3, b3)]
    for layer, (W, b) in enumerate(params):
        acc1, acc2 = _segsum_sc(x1, src1, dst1, x2, src2, dst2, zeros128)
        last = layer == len(params) - 1
        b2d = b.reshape(1, D)
        x1 = _tc_layer(x1, acc1, deg1, W, b2d, last)
        x2 = _tc_layer(x2, acc2, deg2, W, b2d, last)
    return (x1, x2)


# speedup vs baseline: 5.2335x; 1.5766x over previous
import functools

import jax
import jax.numpy as jnp
from jax import lax
from jax.experimental import pallas as pl
from jax.experimental.pallas import tpu as pltpu
from jax.experimental.pallas import tpu_sc as plsc

N = 10000
E = 320000
D = 128

NUM_TILES = 16
NPAD = 10240
ROWS_PER_TILE = NPAD // NUM_TILES
EDGES_PER_TILE = E // NUM_TILES
CHUNK = 120
NUM_CHUNKS = 168
PAD_EDGES = NUM_CHUNKS * CHUNK - EDGES_PER_TILE
NBUF = 3
NIDX = 6
UNROLL = 6
NUM_ITERS = NUM_CHUNKS // UNROLL
DUMMY_DST = NPAD - 1

_MESH = plsc.VectorSubcoreMesh(core_axis_name="c", subcore_axis_name="s")


@functools.partial(
    pl.kernel,
    mesh=_MESH,
    out_type=[
        jax.ShapeDtypeStruct((NPAD, D), jnp.float32),
        jax.ShapeDtypeStruct((NPAD, D), jnp.float32),
    ],
    scratch_types=[
        pltpu.VMEM((NIDX, CHUNK), jnp.int32),
        pltpu.VMEM((NIDX, CHUNK), jnp.int32),
        pltpu.VMEM((NBUF, CHUNK, D), jnp.float32),
        pltpu.VMEM_SHARED((NPAD, D), jnp.float32),
        pltpu.SemaphoreType.DMA((NIDX,)),
        pltpu.SemaphoreType.DMA((NBUF,)),
        pltpu.SemaphoreType.DMA((NBUF,)),
    ],
)
def _segsum_sc(x1_hbm, src1_hbm, dst1_hbm, x2_hbm, src2_hbm, dst2_hbm,
               zeros_hbm, acc1_hbm, acc2_hbm,
               sidx, didx, bufs, acc_sh, semi, semg, sems):
    c = lax.axis_index("c")
    s = lax.axis_index("s")
    row0 = s * ROWS_PER_TILE

    pltpu.sync_copy(zeros_hbm, acc_sh.at[pl.ds(row0, ROWS_PER_TILE)])
    plsc.subcore_barrier()

    def _edge_loop(x_hbm, src_hbm, dst_hbm):
        base = s * NUM_CHUNKS * CHUNK

        def start_idx(i, k):
            off = base + i * CHUNK
            pltpu.async_copy(src_hbm.at[pl.ds(off, CHUNK)], sidx.at[k],
                             semi.at[k])
            pltpu.async_copy(dst_hbm.at[pl.ds(off, CHUNK)], didx.at[k],
                             semi.at[k])

        def wait_idx(i, k):
            off = base + i * CHUNK
            pltpu.make_async_copy(src_hbm.at[pl.ds(off, CHUNK)], sidx.at[k],
                                  semi.at[k]).wait()
            pltpu.make_async_copy(dst_hbm.at[pl.ds(off, CHUNK)], didx.at[k],
                                  semi.at[k]).wait()

        def wait_scatter(j, k):
            pltpu.make_async_copy(bufs.at[j], acc_sh.at[didx.at[k]],
                                  sems.at[j]).wait()

        for u in range(NBUF):
            start_idx(u, u)

        def body(q, carry):
            for u in range(UNROLL):
                i = UNROLL * q + u
                j = u % NBUF
                k = u
                kpre = (u + NBUF) % NIDX

                if u < NBUF:
                    @pl.when(q > 0)
                    def _():
                        wait_scatter(j, kpre)
                else:
                    wait_scatter(j, kpre)

                if u < NBUF:
                    start_idx(i + NBUF, kpre)
                else:
                    @pl.when(q < NUM_ITERS - 1)
                    def _():
                        start_idx(i + NBUF, kpre)

                wait_idx(i, k)
                pltpu.async_copy(x_hbm.at[sidx.at[k]], bufs.at[j], semg.at[j])
                pltpu.make_async_copy(x_hbm.at[sidx.at[k]], bufs.at[j],
                                      semg.at[j]).wait()
                pltpu.async_copy(bufs.at[j], acc_sh.at[didx.at[k]],
                                 sems.at[j], add=True)
            return carry

        lax.fori_loop(0, NUM_ITERS, body, 0)

        for u in range(NBUF, UNROLL):
            wait_scatter(u % NBUF, u)

    @pl.when(c == 0)
    def _():
        _edge_loop(x1_hbm, src1_hbm, dst1_hbm)

    @pl.when(c == 1)
    def _():
        _edge_loop(x2_hbm, src2_hbm, dst2_hbm)

    plsc.subcore_barrier()

    @pl.when(c == 0)
    def _():
        pltpu.sync_copy(acc_sh.at[pl.ds(row0, ROWS_PER_TILE)],
                        acc1_hbm.at[pl.ds(row0, ROWS_PER_TILE)])

    @pl.when(c == 1)
    def _():
        pltpu.sync_copy(acc_sh.at[pl.ds(row0, ROWS_PER_TILE)],
                        acc2_hbm.at[pl.ds(row0, ROWS_PER_TILE)])


@functools.partial(
    pl.kernel,
    mesh=_MESH,
    out_type=[
        jax.ShapeDtypeStruct((NPAD, 16), jnp.float32),
        jax.ShapeDtypeStruct((NPAD, 16), jnp.float32),
    ],
    scratch_types=[
        pltpu.VMEM((CHUNK,), jnp.int32),
        pltpu.VMEM((CHUNK, 16), jnp.float32),
        pltpu.VMEM_SHARED((NPAD, 16), jnp.float32),
    ],
)
def _degree_sc(dst1_hbm, dst2_hbm, ones_hbm, zeros_hbm,
               deg1_hbm, deg2_hbm, dst_v, ones_v, deg_sh):
    c = lax.axis_index("c")
    s = lax.axis_index("s")
    row0 = s * ROWS_PER_TILE

    pltpu.sync_copy(ones_hbm, ones_v)
    pltpu.sync_copy(zeros_hbm, deg_sh.at[pl.ds(row0, ROWS_PER_TILE)])
    plsc.subcore_barrier()

    def _edge_loop(dst_hbm):
        base = s * NUM_CHUNKS * CHUNK

        def body(i, carry):
            pltpu.sync_copy(dst_hbm.at[pl.ds(base + i * CHUNK, CHUNK)], dst_v)
            pltpu.sync_copy(ones_v, deg_sh.at[dst_v], add=True)
            return carry

        lax.fori_loop(0, NUM_CHUNKS, body, 0)

    @pl.when(c == 0)
    def _():
        _edge_loop(dst1_hbm)

    @pl.when(c == 1)
    def _():
        _edge_loop(dst2_hbm)

    plsc.subcore_barrier()

    @pl.when(c == 0)
    def _():
        pltpu.sync_copy(deg_sh.at[pl.ds(row0, ROWS_PER_TILE)],
                        deg1_hbm.at[pl.ds(row0, ROWS_PER_TILE)])

    @pl.when(c == 1)
    def _():
        pltpu.sync_copy(deg_sh.at[pl.ds(row0, ROWS_PER_TILE)],
                        deg2_hbm.at[pl.ds(row0, ROWS_PER_TILE)])


_TC_BLOCK = 2000


def _tc_layer_body(x_ref, acc_ref, deg_ref, w_ref, b_ref, o_ref, *, last):
    t = jnp.dot(acc_ref[...], w_ref[...],
                preferred_element_type=jnp.float32,
                precision=lax.Precision.HIGHEST)
    t = t + deg_ref[:, 0:1] * b_ref[...]
    t = x_ref[...] + jnp.maximum(t, 0.0)
    if last:
        nrm = jnp.sqrt(jnp.sum(t * t, axis=1, keepdims=True))
        t = t / jnp.maximum(nrm, 1e-12)
    o_ref[...] = t


def _tc_layer(x, acc, deg, W, b2d, last):
    grid = (N // _TC_BLOCK,)
    return pl.pallas_call(
        functools.partial(_tc_layer_body, last=last),
        grid=grid,
        in_specs=[
            pl.BlockSpec((_TC_BLOCK, D), lambda i: (i, 0)),
            pl.BlockSpec((_TC_BLOCK, D), lambda i: (i, 0)),
            pl.BlockSpec((_TC_BLOCK, 16), lambda i: (i, 0)),
            pl.BlockSpec((D, D), lambda i: (0, 0)),
            pl.BlockSpec((1, D), lambda i: (0, 0)),
        ],
        out_specs=pl.BlockSpec((_TC_BLOCK, D), lambda i: (i, 0)),
        out_shape=jax.ShapeDtypeStruct((N, D), jnp.float32),
    )(x, acc, deg, W, b2d)


def kernel(x1, edge_index1, x2, edge_index2,
           W0, b0, W1, b1, W2, b2, W3, b3):
    def _pad_idx(v, fill):
        t = v.astype(jnp.int32).reshape(NUM_TILES, EDGES_PER_TILE)
        t = jnp.pad(t, ((0, 0), (0, PAD_EDGES)), constant_values=fill)
        return t.reshape(NUM_TILES * NUM_CHUNKS * CHUNK)

    src1 = _pad_idx(edge_index1[0], 0)
    dst1 = _pad_idx(edge_index1[1], DUMMY_DST)
    src2 = _pad_idx(edge_index2[0], 0)
    dst2 = _pad_idx(edge_index2[1], DUMMY_DST)

    zeros128 = jnp.zeros((ROWS_PER_TILE, D), jnp.float32)
    zeros16 = jnp.zeros((ROWS_PER_TILE, 16), jnp.float32)
    ones16 = jnp.ones((CHUNK, 16), jnp.float32)

    deg1, deg2 = _degree_sc(dst1, dst2, ones16, zeros16)

    params = [(W0, b0), (W1, b1), (W2, b2), (W3, b3)]
    for layer, (W, b) in enumerate(params):
        acc1, acc2 = _segsum_sc(x1, src1, dst1, x2, src2, dst2, zeros128)
        last = layer == len(params) - 1
        b2d = b.reshape(1, D)
        x1 = _tc_layer(x1, acc1, deg1, W, b2d, last)
        x2 = _tc_layer(x2, acc2, deg2, W, b2d, last)
    return (x1, x2)


# speedup vs baseline: 6.1621x; 1.1774x over previous
import functools

import jax
import jax.numpy as jnp
from jax import lax
from jax.experimental import pallas as pl
from jax.experimental.pallas import tpu as pltpu
from jax.experimental.pallas import tpu_sc as plsc

N = 10000
E = 320000
D = 128

NUM_TILES = 16
NPAD = 10240
ROWS_PER_TILE = NPAD // NUM_TILES
EDGES_PER_TILE = E // NUM_TILES
CHUNK = 120
NUM_CHUNKS = 168
PAD_EDGES = NUM_CHUNKS * CHUNK - EDGES_PER_TILE
NBUF = 3
NIDX = 6
UNROLL = 6
NUM_ITERS = NUM_CHUNKS // UNROLL
DUMMY_DST = NPAD - 1

_MESH = plsc.VectorSubcoreMesh(core_axis_name="c", subcore_axis_name="s")


@functools.partial(
    pl.kernel,
    mesh=_MESH,
    out_type=[
        jax.ShapeDtypeStruct((NPAD, D), jnp.float32),
        jax.ShapeDtypeStruct((NPAD, D), jnp.float32),
    ],
    scratch_types=[
        pltpu.VMEM((NIDX, CHUNK), jnp.int32),
        pltpu.VMEM((NIDX, CHUNK), jnp.int32),
        pltpu.VMEM((NBUF, CHUNK, D), jnp.float32),
        pltpu.VMEM_SHARED((NPAD, D), jnp.float32),
        pltpu.SemaphoreType.DMA((NIDX,)),
        pltpu.SemaphoreType.DMA((NBUF,)),
        pltpu.SemaphoreType.DMA((NBUF,)),
    ],
)
def _segsum_sc(x1_hbm, src1_hbm, dst1_hbm, x2_hbm, src2_hbm, dst2_hbm,
               zeros_hbm, acc1_hbm, acc2_hbm,
               sidx, didx, bufs, acc_sh, semi, semg, sems):
    c = lax.axis_index("c")
    s = lax.axis_index("s")
    row0 = s * ROWS_PER_TILE

    pltpu.sync_copy(zeros_hbm, acc_sh.at[pl.ds(row0, ROWS_PER_TILE)])
    plsc.subcore_barrier()

    def _edge_loop(x_hbm, src_hbm, dst_hbm):
        base = s * NUM_CHUNKS * CHUNK

        def start_idx(i, k):
            off = base + i * CHUNK
            pltpu.async_copy(src_hbm.at[pl.ds(off, CHUNK)], sidx.at[k],
                             semi.at[k])
            pltpu.async_copy(dst_hbm.at[pl.ds(off, CHUNK)], didx.at[k],
                             semi.at[k])

        def wait_idx(i, k):
            off = base + i * CHUNK
            pltpu.make_async_copy(src_hbm.at[pl.ds(off, CHUNK)], sidx.at[k],
                                  semi.at[k]).wait()
            pltpu.make_async_copy(dst_hbm.at[pl.ds(off, CHUNK)], didx.at[k],
                                  semi.at[k]).wait()

        def wait_scatter(j, k):
            pltpu.make_async_copy(bufs.at[j], acc_sh.at[didx.at[k]],
                                  sems.at[j]).wait()

        def start_gather(j, k):
            pltpu.async_copy(x_hbm.at[sidx.at[k]], bufs.at[j], semg.at[j])

        def wait_gather(j, k):
            pltpu.make_async_copy(x_hbm.at[sidx.at[k]], bufs.at[j],
                                  semg.at[j]).wait()

        def start_scatter(j, k):
            pltpu.async_copy(bufs.at[j], acc_sh.at[didx.at[k]],
                             sems.at[j], add=True)

        for u in range(4):
            start_idx(u, u)
        wait_idx(0, 0)
        start_gather(0, 0)

        def body(q, carry):
            for u in range(UNROLL):
                i = UNROLL * q + u
                j = u % NBUF
                k = u
                j1 = (u + 1) % NBUF
                k1 = (u + 1) % NIDX
                k4 = (u + 4) % NIDX

                if u < 2:
                    @pl.when(q > 0)
                    def _():
                        wait_scatter(j1, k4)
                else:
                    wait_scatter(j1, k4)

                if u < 2:
                    start_idx(i + 4, k4)
                else:
                    @pl.when(q < NUM_ITERS - 1)
                    def _():
                        start_idx(i + 4, k4)

                if u < UNROLL - 1:
                    wait_idx(i + 1, k1)
                    start_gather(j1, k1)
                else:
                    @pl.when(q < NUM_ITERS - 1)
                    def _():
                        wait_idx(i + 1, k1)
                        start_gather(j1, k1)

                wait_gather(j, k)
                start_scatter(j, k)
            return carry

        lax.fori_loop(0, NUM_ITERS, body, 0)

        wait_scatter((NUM_CHUNKS - 2) % NBUF, (NUM_CHUNKS - 2) % NIDX)
        wait_scatter((NUM_CHUNKS - 1) % NBUF, (NUM_CHUNKS - 1) % NIDX)

    @pl.when(c == 0)
    def _():
        _edge_loop(x1_hbm, src1_hbm, dst1_hbm)

    @pl.when(c == 1)
    def _():
        _edge_loop(x2_hbm, src2_hbm, dst2_hbm)

    plsc.subcore_barrier()

    @pl.when(c == 0)
    def _():
        pltpu.sync_copy(acc_sh.at[pl.ds(row0, ROWS_PER_TILE)],
                        acc1_hbm.at[pl.ds(row0, ROWS_PER_TILE)])

    @pl.when(c == 1)
    def _():
        pltpu.sync_copy(acc_sh.at[pl.ds(row0, ROWS_PER_TILE)],
                        acc2_hbm.at[pl.ds(row0, ROWS_PER_TILE)])


@functools.partial(
    pl.kernel,
    mesh=_MESH,
    out_type=[
        jax.ShapeDtypeStruct((NPAD, 16), jnp.float32),
        jax.ShapeDtypeStruct((NPAD, 16), jnp.float32),
    ],
    scratch_types=[
        pltpu.VMEM((CHUNK,), jnp.int32),
        pltpu.VMEM((CHUNK, 16), jnp.float32),
        pltpu.VMEM_SHARED((NPAD, 16), jnp.float32),
    ],
)
def _degree_sc(dst1_hbm, dst2_hbm, ones_hbm, zeros_hbm,
               deg1_hbm, deg2_hbm, dst_v, ones_v, deg_sh):
    c = lax.axis_index("c")
    s = lax.axis_index("s")
    row0 = s * ROWS_PER_TILE

    pltpu.sync_copy(ones_hbm, ones_v)
    pltpu.sync_copy(zeros_hbm, deg_sh.at[pl.ds(row0, ROWS_PER_TILE)])
    plsc.subcore_barrier()

    def _edge_loop(dst_hbm):
        base = s * NUM_CHUNKS * CHUNK

        def body(i, carry):
            pltpu.sync_copy(dst_hbm.at[pl.ds(base + i * CHUNK, CHUNK)], dst_v)
            pltpu.sync_copy(ones_v, deg_sh.at[dst_v], add=True)
            return carry

        lax.fori_loop(0, NUM_CHUNKS, body, 0)

    @pl.when(c == 0)
    def _():
        _edge_loop(dst1_hbm)

    @pl.when(c == 1)
    def _():
        _edge_loop(dst2_hbm)

    plsc.subcore_barrier()

    @pl.when(c == 0)
    def _():
        pltpu.sync_copy(deg_sh.at[pl.ds(row0, ROWS_PER_TILE)],
                        deg1_hbm.at[pl.ds(row0, ROWS_PER_TILE)])

    @pl.when(c == 1)
    def _():
        pltpu.sync_copy(deg_sh.at[pl.ds(row0, ROWS_PER_TILE)],
                        deg2_hbm.at[pl.ds(row0, ROWS_PER_TILE)])


_TC_BLOCK = 2000


def _tc_layer_body(x_ref, acc_ref, deg_ref, w_ref, b_ref, o_ref, *, last):
    t = jnp.dot(acc_ref[...], w_ref[...],
                preferred_element_type=jnp.float32,
                precision=lax.Precision.HIGHEST)
    t = t + deg_ref[:, 0:1] * b_ref[...]
    t = x_ref[...] + jnp.maximum(t, 0.0)
    if last:
        nrm = jnp.sqrt(jnp.sum(t * t, axis=1, keepdims=True))
        t = t / jnp.maximum(nrm, 1e-12)
    o_ref[...] = t


def _tc_layer(x, acc, deg, W, b2d, last):
    grid = (N // _TC_BLOCK,)
    return pl.pallas_call(
        functools.partial(_tc_layer_body, last=last),
        grid=grid,
        in_specs=[
            pl.BlockSpec((_TC_BLOCK, D), lambda i: (i, 0)),
            pl.BlockSpec((_TC_BLOCK, D), lambda i: (i, 0)),
            pl.BlockSpec((_TC_BLOCK, 16), lambda i: (i, 0)),
            pl.BlockSpec((D, D), lambda i: (0, 0)),
            pl.BlockSpec((1, D), lambda i: (0, 0)),
        ],
        out_specs=pl.BlockSpec((_TC_BLOCK, D), lambda i: (i, 0)),
        out_shape=jax.ShapeDtypeStruct((N, D), jnp.float32),
    )(x, acc, deg, W, b2d)


def kernel(x1, edge_index1, x2, edge_index2,
           W0, b0, W1, b1, W2, b2, W3, b3):
    def _pad_idx(v, fill):
        t = v.astype(jnp.int32).reshape(NUM_TILES, EDGES_PER_TILE)
        t = jnp.pad(t, ((0, 0), (0, PAD_EDGES)), constant_values=fill)
        return t.reshape(NUM_TILES * NUM_CHUNKS * CHUNK)

    src1 = _pad_idx(edge_index1[0], 0)
    dst1 = _pad_idx(edge_index1[1], DUMMY_DST)
    src2 = _pad_idx(edge_index2[0], 0)
    dst2 = _pad_idx(edge_index2[1], DUMMY_DST)

    zeros128 = jnp.zeros((ROWS_PER_TILE, D), jnp.float32)
    zeros16 = jnp.zeros((ROWS_PER_TILE, 16), jnp.float32)
    ones16 = jnp.ones((CHUNK, 16), jnp.float32)

    deg1, deg2 = _degree_sc(dst1, dst2, ones16, zeros16)

    params = [(W0, b0), (W1, b1), (W2, b2), (W3, b3)]
    for layer, (W, b) in enumerate(params):
        acc1, acc2 = _segsum_sc(x1, src1, dst1, x2, src2, dst2, zeros128)
        last = layer == len(params) - 1
        b2d = b.reshape(1, D)
        x1 = _tc_layer(x1, acc1, deg1, W, b2d, last)
        x2 = _tc_layer(x2, acc2, deg2, W, b2d, last)
    return (x1, x2)


# speedup vs baseline: 6.4016x; 1.0389x over previous
import functools

import jax
import jax.numpy as jnp
from jax import lax
from jax.experimental import pallas as pl
from jax.experimental.pallas import tpu as pltpu
from jax.experimental.pallas import tpu_sc as plsc

N = 10000
E = 320000
D = 128

NUM_TILES = 16
NPAD = 10240
ROWS_PER_TILE = NPAD // NUM_TILES
EDGES_PER_TILE = E // NUM_TILES
CHUNK = 120
NUM_CHUNKS = 168
PAD_EDGES = NUM_CHUNKS * CHUNK - EDGES_PER_TILE
NBUF = 3
NIDX = 6
UNROLL = 6
NUM_ITERS = NUM_CHUNKS // UNROLL
DUMMY_DST = NPAD - 1

_MESH = plsc.VectorSubcoreMesh(core_axis_name="c", subcore_axis_name="s")


@functools.partial(
    pl.kernel,
    mesh=_MESH,
    out_type=[
        jax.ShapeDtypeStruct((2, NPAD, D), jnp.float32),
    ],
    scratch_types=[
        pltpu.VMEM((NIDX, CHUNK), jnp.int32),
        pltpu.VMEM((NIDX, CHUNK), jnp.int32),
        pltpu.VMEM((NBUF, CHUNK, D), jnp.float32),
        pltpu.VMEM_SHARED((NPAD, D), jnp.float32),
        pltpu.SemaphoreType.DMA((NIDX,)),
        pltpu.SemaphoreType.DMA((NBUF,)),
        pltpu.SemaphoreType.DMA((NBUF,)),
    ],
)
def _segsum_sc(xcat_hbm, src_hbm, dst_hbm, zeros_hbm, acc_hbm,
               sidx, didx, bufs, acc_sh, semi, semg, sems):
    c = lax.axis_index("c")
    s = lax.axis_index("s")
    row0 = s * ROWS_PER_TILE
    base = (c * NUM_TILES + s) * NUM_CHUNKS * CHUNK

    pltpu.sync_copy(zeros_hbm, acc_sh.at[pl.ds(row0, ROWS_PER_TILE)])
    plsc.subcore_barrier()

    def start_idx(i, k):
        off = base + i * CHUNK
        pltpu.async_copy(src_hbm.at[pl.ds(off, CHUNK)], sidx.at[k], semi.at[k])
        pltpu.async_copy(dst_hbm.at[pl.ds(off, CHUNK)], didx.at[k], semi.at[k])

    def wait_idx(i, k):
        off = base + i * CHUNK
        pltpu.make_async_copy(src_hbm.at[pl.ds(off, CHUNK)], sidx.at[k],
                              semi.at[k]).wait()
        pltpu.make_async_copy(dst_hbm.at[pl.ds(off, CHUNK)], didx.at[k],
                              semi.at[k]).wait()

    def start_gather(j, k):
        pltpu.async_copy(xcat_hbm.at[sidx.at[k]], bufs.at[j], semg.at[j])

    def wait_gather(j, k):
        pltpu.make_async_copy(xcat_hbm.at[sidx.at[k]], bufs.at[j],
                              semg.at[j]).wait()

    def start_scatter(j, k):
        pltpu.async_copy(bufs.at[j], acc_sh.at[didx.at[k]], sems.at[j],
                         add=True)

    def wait_scatter(j, k):
        pltpu.make_async_copy(bufs.at[j], acc_sh.at[didx.at[k]],
                              sems.at[j]).wait()

    for u in range(4):
        start_idx(u, u)
    wait_idx(0, 0)
    start_gather(0, 0)

    def body(q, carry):
        for u in range(UNROLL):
            i = UNROLL * q + u
            j = u % NBUF
            k = u
            j1 = (u + 1) % NBUF
            k1 = (u + 1) % NIDX
            k4 = (u + 4) % NIDX

            if u < 2:
                @pl.when(q > 0)
                def _():
                    wait_scatter(j1, k4)
            else:
                wait_scatter(j1, k4)

            if u < 2:
                start_idx(i + 4, k4)
            else:
                @pl.when(q < NUM_ITERS - 1)
                def _():
                    start_idx(i + 4, k4)

            if u < UNROLL - 1:
                wait_idx(i + 1, k1)
                start_gather(j1, k1)
            else:
                @pl.when(q < NUM_ITERS - 1)
                def _():
                    wait_idx(i + 1, k1)
                    start_gather(j1, k1)

            wait_gather(j, k)
            start_scatter(j, k)
        return carry

    lax.fori_loop(0, NUM_ITERS, body, 0)

    wait_scatter((NUM_CHUNKS - 2) % NBUF, (NUM_CHUNKS - 2) % NIDX)
    wait_scatter((NUM_CHUNKS - 1) % NBUF, (NUM_CHUNKS - 1) % NIDX)

    plsc.subcore_barrier()
    pltpu.sync_copy(acc_sh.at[pl.ds(row0, ROWS_PER_TILE)],
                    acc_hbm.at[c, pl.ds(row0, ROWS_PER_TILE)])


@functools.partial(
    pl.kernel,
    mesh=_MESH,
    out_type=[
        jax.ShapeDtypeStruct((2, NPAD, 16), jnp.float32),
    ],
    scratch_types=[
        pltpu.VMEM((4, CHUNK), jnp.int32),
        pltpu.VMEM((CHUNK, 16), jnp.float32),
        pltpu.VMEM_SHARED((NPAD, 16), jnp.float32),
        pltpu.SemaphoreType.DMA((4,)),
        pltpu.SemaphoreType.DMA((2,)),
    ],
)
def _degree_sc(dst_hbm, ones_hbm, zeros_hbm, deg_hbm,
               didx, ones_v, deg_sh, semi, sems):
    c = lax.axis_index("c")
    s = lax.axis_index("s")
    row0 = s * ROWS_PER_TILE
    base = (c * NUM_TILES + s) * NUM_CHUNKS * CHUNK

    pltpu.sync_copy(ones_hbm, ones_v)
    pltpu.sync_copy(zeros_hbm, deg_sh.at[pl.ds(row0, ROWS_PER_TILE)])
    plsc.subcore_barrier()

    def start_idx(i, k):
        pltpu.async_copy(dst_hbm.at[pl.ds(base + i * CHUNK, CHUNK)],
                         didx.at[k], semi.at[k])

    def wait_idx(i, k):
        pltpu.make_async_copy(dst_hbm.at[pl.ds(base + i * CHUNK, CHUNK)],
                              didx.at[k], semi.at[k]).wait()

    def start_scatter(j, k):
        pltpu.async_copy(ones_v, deg_sh.at[didx.at[k]], sems.at[j], add=True)

    def wait_scatter(j, k):
        pltpu.make_async_copy(ones_v, deg_sh.at[didx.at[k]],
                              sems.at[j]).wait()

    start_idx(0, 0)
    start_idx(1, 1)

    def body(q, carry):
        for u in range(4):
            i = 4 * q + u
            j = u % 2
            k = u
            k2 = (u + 2) % 4

            if u < 2:
                @pl.when(q > 0)
                def _():
                    wait_scatter(j, k2)
                start_idx(i + 2, k2)
            else:
                wait_scatter(j, k2)

                @pl.when(q < NUM_CHUNKS // 4 - 1)
                def _():
                    start_idx(i + 2, k2)

            wait_idx(i, k)
            start_scatter(j, k)
        return carry

    lax.fori_loop(0, NUM_CHUNKS // 4, body, 0)

    wait_scatter((NUM_CHUNKS - 2) % 2, (NUM_CHUNKS - 2) % 4)
    wait_scatter((NUM_CHUNKS - 1) % 2, (NUM_CHUNKS - 1) % 4)

    plsc.subcore_barrier()
    pltpu.sync_copy(deg_sh.at[pl.ds(row0, ROWS_PER_TILE)],
                    deg_hbm.at[c, pl.ds(row0, ROWS_PER_TILE)])


_TC_BLOCK = 2000
_BLOCKS_PER_GRAPH = N // _TC_BLOCK


def _tc_layer_body(x_ref, acc_ref, deg_ref, w_ref, b_ref, o_ref, *, last):
    t = jnp.dot(acc_ref[...][0], w_ref[...],
                preferred_element_type=jnp.float32,
                precision=lax.Precision.HIGHEST)
    t = t + deg_ref[...][0][:, 0:1] * b_ref[...]
    t = x_ref[...] + jnp.maximum(t, 0.0)
    if last:
        nrm = jnp.sqrt(jnp.sum(t * t, axis=1, keepdims=True))
        t = t / jnp.maximum(nrm, 1e-12)
    o_ref[...] = t


def _tc_layer(xcat, acccat, degcat, W, b2d, last):
    grid = (2 * _BLOCKS_PER_GRAPH,)
    return pl.pallas_call(
        functools.partial(_tc_layer_body, last=last),
        grid=grid,
        in_specs=[
            pl.BlockSpec((_TC_BLOCK, D),
                         lambda i: (i, 0)),
            pl.BlockSpec((1, _TC_BLOCK, D),
                         lambda i: (i // _BLOCKS_PER_GRAPH,
                                    i % _BLOCKS_PER_GRAPH, 0)),
            pl.BlockSpec((1, _TC_BLOCK, 16),
                         lambda i: (i // _BLOCKS_PER_GRAPH,
                                    i % _BLOCKS_PER_GRAPH, 0)),
            pl.BlockSpec((D, D), lambda i: (0, 0)),
            pl.BlockSpec((1, D), lambda i: (0, 0)),
        ],
        out_specs=pl.BlockSpec((_TC_BLOCK, D), lambda i: (i, 0)),
        out_shape=jax.ShapeDtypeStruct((2 * N, D), jnp.float32),
    )(xcat, acccat, degcat, W, b2d)


def kernel(x1, edge_index1, x2, edge_index2,
           W0, b0, W1, b1, W2, b2, W3, b3):
    def _pad_idx(v, fill):
        t = v.astype(jnp.int32).reshape(NUM_TILES, EDGES_PER_TILE)
        t = jnp.pad(t, ((0, 0), (0, PAD_EDGES)), constant_values=fill)
        return t.reshape(NUM_TILES * NUM_CHUNKS * CHUNK)

    srccat = jnp.concatenate([
        _pad_idx(edge_index1[0], 0),
        _pad_idx(edge_index2[0] + N, N),
    ])
    dstcat = jnp.concatenate([
        _pad_idx(edge_index1[1], DUMMY_DST),
        _pad_idx(edge_index2[1], DUMMY_DST),
    ])
    xcat = jnp.concatenate([x1, x2], axis=0)

    zeros128 = jnp.zeros((ROWS_PER_TILE, D), jnp.float32)
    zeros16 = jnp.zeros((ROWS_PER_TILE, 16), jnp.float32)
    ones16 = jnp.ones((CHUNK, 16), jnp.float32)

    degcat, = _degree_sc(dstcat, ones16, zeros16)

    params = [(W0, b0), (W1, b1), (W2, b2), (W3, b3)]
    for layer, (W, b) in enumerate(params):
        acccat, = _segsum_sc(xcat, srccat, dstcat, zeros128)
        last = layer == len(params) - 1
        xcat = _tc_layer(xcat, acccat, degcat, W, b.reshape(1, D), last)
    return (xcat[:N], xcat[N:])
